# Initial kernel scaffold; baseline (speedup 1.0000x reference)
import jax
import jax.numpy as jnp
from jax.experimental import pallas as pl


def kernel(x, conv1_w, conv2_w, gate_w, gate_b, noise_w, noise_b, noise_sample):
    raise NotImplementedError("write your pallas kernel here")


# speedup vs baseline: 4.8470x; 4.8470x over previous
import functools

import jax
import jax.numpy as jnp
from jax.experimental import pallas as pl
from jax.experimental.pallas import tpu as pltpu

_B, _DIM, _H, _W = 8, 96, 56, 56
_E, _K, _FF = 6, 2, 2
_HW = _H * _W
_DFF = _DIM * _FF


def _ndtr(z):
    return 0.5 * (1.0 + jax.lax.erf(z * (1.0 / jnp.sqrt(2.0).astype(jnp.float32))))


def _cv_squared(v):
    n = v.shape[0]
    mean = jnp.sum(v) / n
    var = jnp.sum((v - mean) ** 2) / (n - 1)
    return var / (mean * mean + 1e-10)


def _gating_kernel(x_ref, gate_wt_ref, gate_b_ref, noise_wt_ref, noise_b_ref,
                   noise_t_ref, idx_ref, gate_out_ref, loss_ref):
    mean_cols = []
    max_cols = []
    for b in range(_B):
        xb = x_ref[b]
        mean_cols.append(jnp.mean(xb, axis=1, keepdims=True))
        max_cols.append(jnp.max(xb, axis=1, keepdims=True))
    xv_t = jnp.concatenate(mean_cols + max_cols, axis=1)
    mean_t = xv_t[:, :_B]
    max_t = xv_t[:, _B:]
    xvec_t = jnp.concatenate([mean_t, max_t], axis=0)

    clean_t = jnp.dot(gate_wt_ref[...], xvec_t,
                      preferred_element_type=jnp.float32) + gate_b_ref[...]
    raw_noise_t = jnp.dot(noise_wt_ref[...], xvec_t,
                          preferred_element_type=jnp.float32) + noise_b_ref[...]
    noise_std_t = jax.nn.softplus(raw_noise_t) + 0.01
    noisy_t = clean_t + noise_t_ref[...] * noise_std_t

    row_iota = jax.lax.broadcasted_iota(jnp.int32, (_E, _B), 0)
    rank = jnp.zeros((_E, _B), jnp.int32)
    for j in range(_E):
        vj = noisy_t[j:j + 1, :]
        gt = (vj > noisy_t).astype(jnp.int32)
        eq_first = ((vj == noisy_t) & (j < row_iota)).astype(jnp.int32)
        rank = rank + gt + eq_first

    def take(k, what):
        m = (rank == k).astype(jnp.float32)
        return jnp.sum(m * what, axis=0, keepdims=True)

    v0 = take(0, noisy_t)
    v1 = take(1, noisy_t)
    v2 = take(2, noisy_t)
    i0 = take(0, row_iota.astype(jnp.float32))
    i1 = take(1, row_iota.astype(jnp.float32))

    e1 = jnp.exp(v1 - v0)
    z = 1.0 + e1
    g0 = 1.0 / z
    g1 = e1 / z

    gates_t = (rank == 0).astype(jnp.float32) * g0 + \
              (rank == 1).astype(jnp.float32) * g1
    importance = jnp.sum(gates_t, axis=1, keepdims=True)

    is_in = noisy_t > v2
    prob_in = _ndtr((clean_t - v2) / noise_std_t)
    prob_out = _ndtr((clean_t - v1) / noise_std_t)
    load = jnp.sum(jnp.where(is_in, prob_in, prob_out), axis=1, keepdims=True)

    loss = (_cv_squared(importance) + _cv_squared(load)) * 0.01
    loss_ref[...] = jnp.reshape(loss, (1, 1))
    idx_ref[...] = jnp.concatenate([i0, i1], axis=0).astype(jnp.int32)
    gate_out_ref[...] = jnp.concatenate([g0, g1], axis=0)


def _shift_taps(a, wmask):
    c = a.shape[0]
    out = []
    for ky in range(3):
        for kx in range(3):
            d = (ky - 1) * _W + (kx - 1)
            if d > 0:
                s = jnp.concatenate(
                    [a[:, d:], jnp.zeros((c, d), a.dtype)], axis=1)
            elif d < 0:
                s = jnp.concatenate(
                    [jnp.zeros((c, -d), a.dtype), a[:, :d]], axis=1)
            else:
                s = a
            if kx == 0:
                s = s * wmask[0]
            elif kx == 2:
                s = s * wmask[1]
            out.append(s)
    return out


def _dispatch_kernel(idx_ref, gate_ref, x_ref, w1_ref, w2_ref, out_ref):
    p = pl.program_id(0)
    col = jax.lax.broadcasted_iota(jnp.int32, (1, _HW), 1)
    wpos = jax.lax.rem(col, _W)
    wmask = ((wpos != 0).astype(jnp.float32),
             (wpos != _W - 1).astype(jnp.float32))

    x = x_ref[0]
    stacked1 = jnp.concatenate(_shift_taps(x, wmask), axis=0)
    h = jnp.dot(w1_ref[0], stacked1, preferred_element_type=jnp.float32)
    h = h * _ndtr(h)

    taps2 = _shift_taps(h, wmask)
    acc = jnp.zeros((_DIM, _HW), jnp.float32)
    for ky in range(3):
        st = jnp.concatenate(taps2[3 * ky:3 * ky + 3], axis=0)
        wslice = w2_ref[0][:, 3 * ky * _DFF:(3 * ky + 3) * _DFF]
        acc = acc + jnp.dot(wslice, st, preferred_element_type=jnp.float32)

    contrib = acc * gate_ref[p]

    @pl.when(p % 2 == 0)
    def _():
        out_ref[0] = contrib

    @pl.when(p % 2 == 1)
    def _():
        out_ref[0] = out_ref[0] + contrib


@jax.jit
def kernel(x, conv1_w, conv2_w, gate_w, gate_b, noise_w, noise_b, noise_sample):
    x_flat = x.reshape(_B, _DIM, _HW)

    idx_t, gate_t, loss = pl.pallas_call(
        _gating_kernel,
        out_shape=(
            jax.ShapeDtypeStruct((2, _B), jnp.int32),
            jax.ShapeDtypeStruct((2, _B), jnp.float32),
            jax.ShapeDtypeStruct((1, 1), jnp.float32),
        ),
    )(x_flat,
      gate_w.T, gate_b.reshape(_E, 1),
      noise_w.T, noise_b.reshape(_E, 1),
      noise_sample.T)

    expert_idx = idx_t.T.reshape(_B * _K)
    pair_gate = gate_t.T.reshape(_B * _K)

    w1r = conv1_w.transpose(0, 1, 3, 4, 2).reshape(_E, _DFF, 9 * _DIM)
    w2r = conv2_w.transpose(0, 1, 3, 4, 2).reshape(_E, _DIM, 9 * _DFF)

    grid_spec = pltpu.PrefetchScalarGridSpec(
        num_scalar_prefetch=2,
        grid=(_B * _K,),
        in_specs=[
            pl.BlockSpec((1, _DIM, _HW), lambda p, ei, g: (p // 2, 0, 0)),
            pl.BlockSpec((1, _DFF, 9 * _DIM), lambda p, ei, g: (ei[p], 0, 0)),
            pl.BlockSpec((1, _DIM, 9 * _DFF), lambda p, ei, g: (ei[p], 0, 0)),
        ],
        out_specs=pl.BlockSpec((1, _DIM, _HW), lambda p, ei, g: (p // 2, 0, 0)),
    )

    y = pl.pallas_call(
        _dispatch_kernel,
        grid_spec=grid_spec,
        out_shape=jax.ShapeDtypeStruct((_B, _DIM, _HW), jnp.float32),
        compiler_params=pltpu.CompilerParams(
            dimension_semantics=("arbitrary",),
        ),
    )(expert_idx, pair_gate, x_flat, w1r, w2r)

    return (y.reshape(_B, _DIM, _H, _W), loss[0, 0])


# speedup vs baseline: 4.8884x; 1.0085x over previous
import functools

import jax
import jax.numpy as jnp
from jax.experimental import pallas as pl
from jax.experimental.pallas import tpu as pltpu

_B, _DIM, _H, _W = 8, 96, 56, 56
_E, _K, _FF = 6, 2, 2
_HW = _H * _W
_DFF = _DIM * _FF


def _ndtr(z):
    return 0.5 * (1.0 + jax.lax.erf(z * (1.0 / jnp.sqrt(2.0).astype(jnp.float32))))


def _cv_squared(v):
    n = v.shape[0]
    mean = jnp.sum(v) / n
    var = jnp.sum((v - mean) ** 2) / (n - 1)
    return var / (mean * mean + 1e-10)


def _gating_kernel(x_ref, gate_wt_ref, gate_b_ref, noise_wt_ref, noise_b_ref,
                   noise_t_ref, idx_ref, gate_out_ref, loss_ref):
    mean_cols = []
    max_cols = []
    for b in range(_B):
        xb = x_ref[b]
        mean_cols.append(jnp.mean(xb, axis=1, keepdims=True))
        max_cols.append(jnp.max(xb, axis=1, keepdims=True))
    xv_t = jnp.concatenate(mean_cols + max_cols, axis=1)
    mean_t = xv_t[:, :_B]
    max_t = xv_t[:, _B:]
    xvec_t = jnp.concatenate([mean_t, max_t], axis=0)

    clean_t = jnp.dot(gate_wt_ref[...], xvec_t,
                      preferred_element_type=jnp.float32) + gate_b_ref[...]
    raw_noise_t = jnp.dot(noise_wt_ref[...], xvec_t,
                          preferred_element_type=jnp.float32) + noise_b_ref[...]
    noise_std_t = jax.nn.softplus(raw_noise_t) + 0.01
    noisy_t = clean_t + noise_t_ref[...] * noise_std_t

    row_iota = jax.lax.broadcasted_iota(jnp.int32, (_E, _B), 0)
    rank = jnp.zeros((_E, _B), jnp.int32)
    for j in range(_E):
        vj = noisy_t[j:j + 1, :]
        gt = (vj > noisy_t).astype(jnp.int32)
        eq_first = ((vj == noisy_t) & (j < row_iota)).astype(jnp.int32)
        rank = rank + gt + eq_first

    def take(k, what):
        m = (rank == k).astype(jnp.float32)
        return jnp.sum(m * what, axis=0, keepdims=True)

    v0 = take(0, noisy_t)
    v1 = take(1, noisy_t)
    v2 = take(2, noisy_t)
    i0 = take(0, row_iota.astype(jnp.float32))
    i1 = take(1, row_iota.astype(jnp.float32))

    e1 = jnp.exp(v1 - v0)
    z = 1.0 + e1
    g0 = 1.0 / z
    g1 = e1 / z

    gates_t = (rank == 0).astype(jnp.float32) * g0 + \
              (rank == 1).astype(jnp.float32) * g1
    importance = jnp.sum(gates_t, axis=1, keepdims=True)

    is_in = noisy_t > v2
    prob_in = _ndtr((clean_t - v2) / noise_std_t)
    prob_out = _ndtr((clean_t - v1) / noise_std_t)
    load = jnp.sum(jnp.where(is_in, prob_in, prob_out), axis=1, keepdims=True)

    loss = (_cv_squared(importance) + _cv_squared(load)) * 0.01
    loss_ref[...] = jnp.reshape(loss, (1, 1))
    idx_ref[...] = jnp.concatenate([i0, i1], axis=0).astype(jnp.int32)
    gate_out_ref[...] = jnp.concatenate([g0, g1], axis=0)


def _shift_taps(a, wmask):
    c = a.shape[0]
    out = []
    for ky in range(3):
        for kx in range(3):
            d = (ky - 1) * _W + (kx - 1)
            if d > 0:
                s = jnp.concatenate(
                    [a[:, d:], jnp.zeros((c, d), a.dtype)], axis=1)
            elif d < 0:
                s = jnp.concatenate(
                    [jnp.zeros((c, -d), a.dtype), a[:, :d]], axis=1)
            else:
                s = a
            if kx == 0:
                s = s * wmask[0]
            elif kx == 2:
                s = s * wmask[1]
            out.append(s.astype(jnp.bfloat16))
    return out


def _dispatch_kernel(idx_ref, gate_ref, x_ref, w1_ref, w2_ref, out_ref):
    p = pl.program_id(0)
    col = jax.lax.broadcasted_iota(jnp.int32, (1, _HW), 1)
    wpos = jax.lax.rem(col, _W)
    wmask = ((wpos != 0).astype(jnp.float32),
             (wpos != _W - 1).astype(jnp.float32))

    x = x_ref[0]
    stacked1 = jnp.concatenate(_shift_taps(x, wmask), axis=0)
    h = jnp.dot(w1_ref[0], stacked1, preferred_element_type=jnp.float32)
    h = h * _ndtr(h)

    taps2 = _shift_taps(h, wmask)
    acc = jnp.zeros((_DIM, _HW), jnp.float32)
    for ky in range(3):
        st = jnp.concatenate(taps2[3 * ky:3 * ky + 3], axis=0)
        wslice = w2_ref[0][:, 3 * ky * _DFF:(3 * ky + 3) * _DFF]
        acc = acc + jnp.dot(wslice, st, preferred_element_type=jnp.float32)

    contrib = acc * gate_ref[p]

    @pl.when(p % 2 == 0)
    def _():
        out_ref[0] = contrib

    @pl.when(p % 2 == 1)
    def _():
        out_ref[0] = out_ref[0] + contrib


@jax.jit
def kernel(x, conv1_w, conv2_w, gate_w, gate_b, noise_w, noise_b, noise_sample):
    x_flat = x.reshape(_B, _DIM, _HW)

    idx_t, gate_t, loss = pl.pallas_call(
        _gating_kernel,
        out_shape=(
            jax.ShapeDtypeStruct((2, _B), jnp.int32),
            jax.ShapeDtypeStruct((2, _B), jnp.float32),
            jax.ShapeDtypeStruct((1, 1), jnp.float32),
        ),
    )(x_flat,
      gate_w.T, gate_b.reshape(_E, 1),
      noise_w.T, noise_b.reshape(_E, 1),
      noise_sample.T)

    expert_idx = idx_t.T.reshape(_B * _K)
    pair_gate = gate_t.T.reshape(_B * _K)

    w1r = conv1_w.transpose(0, 1, 3, 4, 2).reshape(
        _E, _DFF, 9 * _DIM).astype(jnp.bfloat16)
    w2r = conv2_w.transpose(0, 1, 3, 4, 2).reshape(
        _E, _DIM, 9 * _DFF).astype(jnp.bfloat16)

    grid_spec = pltpu.PrefetchScalarGridSpec(
        num_scalar_prefetch=2,
        grid=(_B * _K,),
        in_specs=[
            pl.BlockSpec((1, _DIM, _HW), lambda p, ei, g: (p // 2, 0, 0)),
            pl.BlockSpec((1, _DFF, 9 * _DIM), lambda p, ei, g: (ei[p], 0, 0)),
            pl.BlockSpec((1, _DIM, 9 * _DFF), lambda p, ei, g: (ei[p], 0, 0)),
        ],
        out_specs=pl.BlockSpec((1, _DIM, _HW), lambda p, ei, g: (p // 2, 0, 0)),
    )

    y = pl.pallas_call(
        _dispatch_kernel,
        grid_spec=grid_spec,
        out_shape=jax.ShapeDtypeStruct((_B, _DIM, _HW), jnp.float32),
        compiler_params=pltpu.CompilerParams(
            dimension_semantics=("arbitrary",),
        ),
    )(expert_idx, pair_gate, x_flat, w1r, w2r)

    return (y.reshape(_B, _DIM, _H, _W), loss[0, 0])


# speedup vs baseline: 5.9309x; 1.2133x over previous
import functools

import jax
import jax.numpy as jnp
from jax.experimental import pallas as pl
from jax.experimental.pallas import tpu as pltpu

_B, _DIM, _H, _W = 8, 96, 56, 56
_E, _K, _FF = 6, 2, 2
_HW = _H * _W
_DFF = _DIM * _FF


def _ndtr(z):
    return 0.5 * (1.0 + jax.lax.erf(z * (1.0 / jnp.sqrt(2.0).astype(jnp.float32))))


def _cv_squared(v):
    n = v.shape[0]
    mean = jnp.sum(v) / n
    var = jnp.sum((v - mean) ** 2) / (n - 1)
    return var / (mean * mean + 1e-10)


def _gating_kernel(x_ref, gate_wt_ref, gate_b_ref, noise_wt_ref, noise_b_ref,
                   noise_t_ref, idx_ref, gate_out_ref, loss_ref):
    mean_cols = []
    max_cols = []
    for b in range(_B):
        xb = x_ref[b]
        mean_cols.append(jnp.mean(xb, axis=1, keepdims=True))
        max_cols.append(jnp.max(xb, axis=1, keepdims=True))
    xv_t = jnp.concatenate(mean_cols + max_cols, axis=1)
    mean_t = xv_t[:, :_B]
    max_t = xv_t[:, _B:]
    xvec_t = jnp.concatenate([mean_t, max_t], axis=0)

    clean_t = jnp.dot(gate_wt_ref[...], xvec_t,
                      preferred_element_type=jnp.float32) + gate_b_ref[...]
    raw_noise_t = jnp.dot(noise_wt_ref[...], xvec_t,
                          preferred_element_type=jnp.float32) + noise_b_ref[...]
    noise_std_t = jax.nn.softplus(raw_noise_t) + 0.01
    noisy_t = clean_t + noise_t_ref[...] * noise_std_t

    row_iota = jax.lax.broadcasted_iota(jnp.int32, (_E, _B), 0)
    rank = jnp.zeros((_E, _B), jnp.int32)
    for j in range(_E):
        vj = noisy_t[j:j + 1, :]
        gt = (vj > noisy_t).astype(jnp.int32)
        eq_first = ((vj == noisy_t) & (j < row_iota)).astype(jnp.int32)
        rank = rank + gt + eq_first

    def take(k, what):
        m = (rank == k).astype(jnp.float32)
        return jnp.sum(m * what, axis=0, keepdims=True)

    v0 = take(0, noisy_t)
    v1 = take(1, noisy_t)
    v2 = take(2, noisy_t)
    i0 = take(0, row_iota.astype(jnp.float32))
    i1 = take(1, row_iota.astype(jnp.float32))

    e1 = jnp.exp(v1 - v0)
    z = 1.0 + e1
    g0 = 1.0 / z
    g1 = e1 / z

    gates_t = (rank == 0).astype(jnp.float32) * g0 + \
              (rank == 1).astype(jnp.float32) * g1
    importance = jnp.sum(gates_t, axis=1, keepdims=True)

    is_in = noisy_t > v2
    prob_in = _ndtr((clean_t - v2) / noise_std_t)
    prob_out = _ndtr((clean_t - v1) / noise_std_t)
    load = jnp.sum(jnp.where(is_in, prob_in, prob_out), axis=1, keepdims=True)

    loss = (_cv_squared(importance) + _cv_squared(load)) * 0.01
    loss_ref[...] = jnp.reshape(loss, (1, 1))
    idx_ref[...] = jnp.concatenate([i0, i1], axis=0).astype(jnp.int32)
    gate_out_ref[...] = jnp.concatenate([g0, g1], axis=0)


def _shift_taps(a, wmask):
    c = a.shape[0]
    out = []
    for ky in range(3):
        for kx in range(3):
            d = (ky - 1) * _W + (kx - 1)
            if d > 0:
                s = jnp.concatenate(
                    [a[:, d:], jnp.zeros((c, d), a.dtype)], axis=1)
            elif d < 0:
                s = jnp.concatenate(
                    [jnp.zeros((c, -d), a.dtype), a[:, :d]], axis=1)
            else:
                s = a
            if kx == 0:
                s = s * wmask[0]
            elif kx == 2:
                s = s * wmask[1]
            out.append(s.astype(jnp.bfloat16))
    return out


def _dispatch_kernel(idx_ref, gate_ref, x_ref, w1_ref, w2_ref, out_ref):
    p = pl.program_id(0)
    col = jax.lax.broadcasted_iota(jnp.int32, (1, _HW), 1)
    wpos = jax.lax.rem(col, _W)
    wmask = ((wpos != 0).astype(jnp.float32),
             (wpos != _W - 1).astype(jnp.float32))

    x = x_ref[0]
    stacked1 = jnp.concatenate(_shift_taps(x, wmask), axis=0)
    h = jnp.dot(w1_ref[0], stacked1, preferred_element_type=jnp.float32)
    h = h * _ndtr(h)

    res = jnp.dot(w2_ref[0], h.astype(jnp.bfloat16),
                  preferred_element_type=jnp.float32)
    acc = None
    for ky in range(3):
        for kx in range(3):
            t = ky * 3 + kx
            d = (ky - 1) * _W + (kx - 1)
            blk = res[t * _DIM:(t + 1) * _DIM]
            if d > 0:
                s = jnp.concatenate(
                    [blk[:, d:], jnp.zeros((_DIM, d), jnp.float32)], axis=1)
            elif d < 0:
                s = jnp.concatenate(
                    [jnp.zeros((_DIM, -d), jnp.float32), blk[:, :d]], axis=1)
            else:
                s = blk
            if kx == 0:
                s = s * wmask[0]
            elif kx == 2:
                s = s * wmask[1]
            acc = s if acc is None else acc + s

    contrib = acc * gate_ref[p]

    @pl.when(p % 2 == 0)
    def _():
        out_ref[0] = contrib

    @pl.when(p % 2 == 1)
    def _():
        out_ref[0] = out_ref[0] + contrib


@jax.jit
def kernel(x, conv1_w, conv2_w, gate_w, gate_b, noise_w, noise_b, noise_sample):
    x_flat = x.reshape(_B, _DIM, _HW)

    idx_t, gate_t, loss = pl.pallas_call(
        _gating_kernel,
        out_shape=(
            jax.ShapeDtypeStruct((2, _B), jnp.int32),
            jax.ShapeDtypeStruct((2, _B), jnp.float32),
            jax.ShapeDtypeStruct((1, 1), jnp.float32),
        ),
    )(x_flat,
      gate_w.T, gate_b.reshape(_E, 1),
      noise_w.T, noise_b.reshape(_E, 1),
      noise_sample.T)

    expert_idx = idx_t.T.reshape(_B * _K)
    pair_gate = gate_t.T.reshape(_B * _K)

    w1r = conv1_w.transpose(0, 1, 3, 4, 2).reshape(
        _E, _DFF, 9 * _DIM).astype(jnp.bfloat16)
    w2r = conv2_w.transpose(0, 3, 4, 1, 2).reshape(
        _E, 9 * _DIM, _DFF).astype(jnp.bfloat16)

    grid_spec = pltpu.PrefetchScalarGridSpec(
        num_scalar_prefetch=2,
        grid=(_B * _K,),
        in_specs=[
            pl.BlockSpec((1, _DIM, _HW), lambda p, ei, g: (p // 2, 0, 0)),
            pl.BlockSpec((1, _DFF, 9 * _DIM), lambda p, ei, g: (ei[p], 0, 0)),
            pl.BlockSpec((1, 9 * _DIM, _DFF), lambda p, ei, g: (ei[p], 0, 0)),
        ],
        out_specs=pl.BlockSpec((1, _DIM, _HW), lambda p, ei, g: (p // 2, 0, 0)),
    )

    y = pl.pallas_call(
        _dispatch_kernel,
        grid_spec=grid_spec,
        out_shape=jax.ShapeDtypeStruct((_B, _DIM, _HW), jnp.float32),
        compiler_params=pltpu.CompilerParams(
            dimension_semantics=("arbitrary",),
        ),
    )(expert_idx, pair_gate, x_flat, w1r, w2r)

    return (y.reshape(_B, _DIM, _H, _W), loss[0, 0])


# speedup vs baseline: 5.9530x; 1.0037x over previous
import functools

import jax
import jax.numpy as jnp
from jax.experimental import pallas as pl
from jax.experimental.pallas import tpu as pltpu

_B, _DIM, _H, _W = 8, 96, 56, 56
_E, _K, _FF = 6, 2, 2
_HW = _H * _W
_DFF = _DIM * _FF


def _ndtr(z):
    return 0.5 * (1.0 + jax.lax.erf(z * (1.0 / jnp.sqrt(2.0).astype(jnp.float32))))


def _cv_squared(v):
    n = v.shape[0]
    mean = jnp.sum(v) / n
    var = jnp.sum((v - mean) ** 2) / (n - 1)
    return var / (mean * mean + 1e-10)


def _gating_kernel(x_ref, gate_wt_ref, gate_b_ref, noise_wt_ref, noise_b_ref,
                   noise_t_ref, idx_ref, gate_out_ref, loss_ref):
    mean_cols = []
    max_cols = []
    for b in range(_B):
        xb = x_ref[b]
        mean_cols.append(jnp.mean(xb, axis=1, keepdims=True))
        max_cols.append(jnp.max(xb, axis=1, keepdims=True))
    xv_t = jnp.concatenate(mean_cols + max_cols, axis=1)
    mean_t = xv_t[:, :_B]
    max_t = xv_t[:, _B:]
    xvec_t = jnp.concatenate([mean_t, max_t], axis=0)

    clean_t = jnp.dot(gate_wt_ref[...], xvec_t,
                      preferred_element_type=jnp.float32) + gate_b_ref[...]
    raw_noise_t = jnp.dot(noise_wt_ref[...], xvec_t,
                          preferred_element_type=jnp.float32) + noise_b_ref[...]
    noise_std_t = jax.nn.softplus(raw_noise_t) + 0.01
    noisy_t = clean_t + noise_t_ref[...] * noise_std_t

    row_iota = jax.lax.broadcasted_iota(jnp.int32, (_E, _B), 0)
    rank = jnp.zeros((_E, _B), jnp.int32)
    for j in range(_E):
        vj = noisy_t[j:j + 1, :]
        gt = (vj > noisy_t).astype(jnp.int32)
        eq_first = ((vj == noisy_t) & (j < row_iota)).astype(jnp.int32)
        rank = rank + gt + eq_first

    def take(k, what):
        m = (rank == k).astype(jnp.float32)
        return jnp.sum(m * what, axis=0, keepdims=True)

    v0 = take(0, noisy_t)
    v1 = take(1, noisy_t)
    v2 = take(2, noisy_t)
    i0 = take(0, row_iota.astype(jnp.float32))
    i1 = take(1, row_iota.astype(jnp.float32))

    e1 = jnp.exp(v1 - v0)
    z = 1.0 + e1
    g0 = 1.0 / z
    g1 = e1 / z

    gates_t = (rank == 0).astype(jnp.float32) * g0 + \
              (rank == 1).astype(jnp.float32) * g1
    importance = jnp.sum(gates_t, axis=1, keepdims=True)

    is_in = noisy_t > v2
    prob_in = _ndtr((clean_t - v2) / noise_std_t)
    prob_out = _ndtr((clean_t - v1) / noise_std_t)
    load = jnp.sum(jnp.where(is_in, prob_in, prob_out), axis=1, keepdims=True)

    loss = (_cv_squared(importance) + _cv_squared(load)) * 0.01
    loss_ref[...] = jnp.reshape(loss, (1, 1))
    idx_ref[...] = jnp.concatenate([i0, i1], axis=0).astype(jnp.int32)
    gate_out_ref[...] = jnp.concatenate([g0, g1], axis=0)


def _shift_taps(a, wmask):
    c = a.shape[0]
    out = []
    for ky in range(3):
        for kx in range(3):
            d = (ky - 1) * _W + (kx - 1)
            if d > 0:
                s = jnp.concatenate(
                    [a[:, d:], jnp.zeros((c, d), a.dtype)], axis=1)
            elif d < 0:
                s = jnp.concatenate(
                    [jnp.zeros((c, -d), a.dtype), a[:, :d]], axis=1)
            else:
                s = a
            if kx == 0:
                s = s * wmask[0]
            elif kx == 2:
                s = s * wmask[1]
            out.append(s.astype(jnp.bfloat16))
    return out


def _dispatch_kernel(idx_ref, gate_ref, x_ref, w1_ref, w2_ref, out_ref):
    p = pl.program_id(0)
    col = jax.lax.broadcasted_iota(jnp.int32, (1, _HW), 1)
    wpos = jax.lax.rem(col, _W)
    wmask = ((wpos != 0).astype(jnp.float32),
             (wpos != _W - 1).astype(jnp.float32))

    x = x_ref[0]
    stacked1 = jnp.concatenate(_shift_taps(x, wmask), axis=0)
    h = jnp.dot(w1_ref[0], stacked1, preferred_element_type=jnp.float32)
    hb = h.astype(jnp.bfloat16)
    c0 = jnp.bfloat16(0.7978845608028654)
    c1 = jnp.bfloat16(0.044715)
    half = jnp.bfloat16(0.5)
    one = jnp.bfloat16(1.0)
    inner = c0 * (hb + c1 * hb * hb * hb)
    hb = half * hb * (one + jnp.tanh(inner))

    res = jnp.dot(w2_ref[0], hb,
                  preferred_element_type=jnp.float32)
    acc = None
    for ky in range(3):
        for kx in range(3):
            t = ky * 3 + kx
            d = (ky - 1) * _W + (kx - 1)
            blk = res[t * _DIM:(t + 1) * _DIM]
            if d > 0:
                s = jnp.concatenate(
                    [blk[:, d:], jnp.zeros((_DIM, d), jnp.float32)], axis=1)
            elif d < 0:
                s = jnp.concatenate(
                    [jnp.zeros((_DIM, -d), jnp.float32), blk[:, :d]], axis=1)
            else:
                s = blk
            if kx == 0:
                s = s * wmask[0]
            elif kx == 2:
                s = s * wmask[1]
            acc = s if acc is None else acc + s

    contrib = acc * gate_ref[p]

    @pl.when(p % 2 == 0)
    def _():
        out_ref[0] = contrib

    @pl.when(p % 2 == 1)
    def _():
        out_ref[0] = out_ref[0] + contrib


@jax.jit
def kernel(x, conv1_w, conv2_w, gate_w, gate_b, noise_w, noise_b, noise_sample):
    x_flat = x.reshape(_B, _DIM, _HW)

    idx_t, gate_t, loss = pl.pallas_call(
        _gating_kernel,
        out_shape=(
            jax.ShapeDtypeStruct((2, _B), jnp.int32),
            jax.ShapeDtypeStruct((2, _B), jnp.float32),
            jax.ShapeDtypeStruct((1, 1), jnp.float32),
        ),
    )(x_flat,
      gate_w.T, gate_b.reshape(_E, 1),
      noise_w.T, noise_b.reshape(_E, 1),
      noise_sample.T)

    expert_idx = idx_t.T.reshape(_B * _K)
    pair_gate = gate_t.T.reshape(_B * _K)

    w1r = conv1_w.transpose(0, 1, 3, 4, 2).reshape(
        _E, _DFF, 9 * _DIM).astype(jnp.bfloat16)
    w2r = conv2_w.transpose(0, 3, 4, 1, 2).reshape(
        _E, 9 * _DIM, _DFF).astype(jnp.bfloat16)

    grid_spec = pltpu.PrefetchScalarGridSpec(
        num_scalar_prefetch=2,
        grid=(_B * _K,),
        in_specs=[
            pl.BlockSpec((1, _DIM, _HW), lambda p, ei, g: (p // 2, 0, 0)),
            pl.BlockSpec((1, _DFF, 9 * _DIM), lambda p, ei, g: (ei[p], 0, 0)),
            pl.BlockSpec((1, 9 * _DIM, _DFF), lambda p, ei, g: (ei[p], 0, 0)),
        ],
        out_specs=pl.BlockSpec((1, _DIM, _HW), lambda p, ei, g: (p // 2, 0, 0)),
    )

    y = pl.pallas_call(
        _dispatch_kernel,
        grid_spec=grid_spec,
        out_shape=jax.ShapeDtypeStruct((_B, _DIM, _HW), jnp.float32),
        compiler_params=pltpu.CompilerParams(
            dimension_semantics=("arbitrary",),
        ),
    )(expert_idx, pair_gate, x_flat, w1r, w2r)

    return (y.reshape(_B, _DIM, _H, _W), loss[0, 0])


# speedup vs baseline: 8.6107x; 1.4464x over previous
import functools

import jax
import jax.numpy as jnp
from jax.experimental import pallas as pl
from jax.experimental.pallas import tpu as pltpu

_B, _DIM, _H, _W = 8, 96, 56, 56
_E, _K, _FF = 6, 2, 2
_HW = _H * _W
_DFF = _DIM * _FF


def _ndtr(z):
    return 0.5 * (1.0 + jax.lax.erf(z * (1.0 / jnp.sqrt(2.0).astype(jnp.float32))))


def _cv_squared(v):
    n = v.shape[0]
    mean = jnp.sum(v) / n
    var = jnp.sum((v - mean) ** 2) / (n - 1)
    return var / (mean * mean + 1e-10)


def _gating_kernel(x_ref, gate_wt_ref, gate_b_ref, noise_wt_ref, noise_b_ref,
                   noise_t_ref, idx_ref, gate_out_ref, loss_ref):
    mean_cols = []
    max_cols = []
    for b in range(_B):
        xb = x_ref[b]
        mean_cols.append(jnp.mean(xb, axis=1, keepdims=True))
        max_cols.append(jnp.max(xb, axis=1, keepdims=True))
    xv_t = jnp.concatenate(mean_cols + max_cols, axis=1)
    mean_t = xv_t[:, :_B]
    max_t = xv_t[:, _B:]
    xvec_t = jnp.concatenate([mean_t, max_t], axis=0)

    clean_t = jnp.dot(gate_wt_ref[...], xvec_t,
                      preferred_element_type=jnp.float32) + gate_b_ref[...]
    raw_noise_t = jnp.dot(noise_wt_ref[...], xvec_t,
                          preferred_element_type=jnp.float32) + noise_b_ref[...]
    noise_std_t = jax.nn.softplus(raw_noise_t) + 0.01
    noisy_t = clean_t + noise_t_ref[...] * noise_std_t

    row_iota = jax.lax.broadcasted_iota(jnp.int32, (_E, _B), 0)
    rank = jnp.zeros((_E, _B), jnp.int32)
    for j in range(_E):
        vj = noisy_t[j:j + 1, :]
        gt = (vj > noisy_t).astype(jnp.int32)
        eq_first = ((vj == noisy_t) & (j < row_iota)).astype(jnp.int32)
        rank = rank + gt + eq_first

    def take(k, what):
        m = (rank == k).astype(jnp.float32)
        return jnp.sum(m * what, axis=0, keepdims=True)

    v0 = take(0, noisy_t)
    v1 = take(1, noisy_t)
    v2 = take(2, noisy_t)
    i0 = take(0, row_iota.astype(jnp.float32))
    i1 = take(1, row_iota.astype(jnp.float32))

    e1 = jnp.exp(v1 - v0)
    z = 1.0 + e1
    g0 = 1.0 / z
    g1 = e1 / z

    gates_t = (rank == 0).astype(jnp.float32) * g0 + \
              (rank == 1).astype(jnp.float32) * g1
    importance = jnp.sum(gates_t, axis=1, keepdims=True)

    is_in = noisy_t > v2
    prob_in = _ndtr((clean_t - v2) / noise_std_t)
    prob_out = _ndtr((clean_t - v1) / noise_std_t)
    load = jnp.sum(jnp.where(is_in, prob_in, prob_out), axis=1, keepdims=True)

    loss = (_cv_squared(importance) + _cv_squared(load)) * 0.01
    loss_ref[...] = jnp.reshape(loss, (1, 1))
    idx_ref[...] = jnp.concatenate([i0, i1], axis=0).astype(jnp.int32)
    gate_out_ref[...] = jnp.concatenate([g0, g1], axis=0)


def _shift_taps(a, wmask):
    c = a.shape[0]
    out = []
    for ky in range(3):
        for kx in range(3):
            d = (ky - 1) * _W + (kx - 1)
            if d > 0:
                s = jnp.concatenate(
                    [a[:, d:], jnp.zeros((c, d), a.dtype)], axis=1)
            elif d < 0:
                s = jnp.concatenate(
                    [jnp.zeros((c, -d), a.dtype), a[:, :d]], axis=1)
            else:
                s = a
            if kx == 0:
                s = s * wmask[0]
            elif kx == 2:
                s = s * wmask[1]
            out.append(s.astype(jnp.bfloat16))
    return out


def _gelu_scaled_bf16(h, g):
    hb = h.astype(jnp.bfloat16)
    c0 = jnp.bfloat16(0.7978845608028654)
    c1 = jnp.bfloat16(0.044715)
    one = jnp.bfloat16(1.0)
    half_g = (0.5 * g).astype(jnp.bfloat16)
    inner = c0 * (hb + c1 * hb * hb * hb)
    return half_g * hb * (one + jnp.tanh(inner))


def _dispatch_kernel(idx_ref, gate_ref, x_ref, w1a_ref, w1b_ref,
                     w2a_ref, w2b_ref, out_ref):
    s = pl.program_id(0)
    col = jax.lax.broadcasted_iota(jnp.int32, (1, _HW), 1)
    wpos = jax.lax.rem(col, _W)
    wmask = ((wpos != 0).astype(jnp.float32),
             (wpos != _W - 1).astype(jnp.float32))

    x = x_ref[0]
    stacked1 = jnp.concatenate(_shift_taps(x, wmask), axis=0)
    h0 = jnp.dot(w1a_ref[0], stacked1, preferred_element_type=jnp.float32)
    h1 = jnp.dot(w1b_ref[0], stacked1, preferred_element_type=jnp.float32)
    hb0 = _gelu_scaled_bf16(h0, gate_ref[2 * s])
    hb1 = _gelu_scaled_bf16(h1, gate_ref[2 * s + 1])

    hcat = jnp.concatenate([hb0, hb1], axis=0)
    w2cat = jnp.concatenate([w2a_ref[0], w2b_ref[0]], axis=1)
    res = jnp.dot(w2cat, hcat,
                  preferred_element_type=jnp.float32)
    acc = None
    for ky in range(3):
        for kx in range(3):
            t = ky * 3 + kx
            d = (ky - 1) * _W + (kx - 1)
            blk = res[t * _DIM:(t + 1) * _DIM]
            if d > 0:
                sh = jnp.concatenate(
                    [blk[:, d:], jnp.zeros((_DIM, d), jnp.float32)], axis=1)
            elif d < 0:
                sh = jnp.concatenate(
                    [jnp.zeros((_DIM, -d), jnp.float32), blk[:, :d]], axis=1)
            else:
                sh = blk
            if kx == 0:
                sh = sh * wmask[0]
            elif kx == 2:
                sh = sh * wmask[1]
            acc = sh if acc is None else acc + sh

    out_ref[0] = acc


@jax.jit
def kernel(x, conv1_w, conv2_w, gate_w, gate_b, noise_w, noise_b, noise_sample):
    x_flat = x.reshape(_B, _DIM, _HW)

    idx_t, gate_t, loss = pl.pallas_call(
        _gating_kernel,
        out_shape=(
            jax.ShapeDtypeStruct((2, _B), jnp.int32),
            jax.ShapeDtypeStruct((2, _B), jnp.float32),
            jax.ShapeDtypeStruct((1, 1), jnp.float32),
        ),
    )(x_flat,
      gate_w.T, gate_b.reshape(_E, 1),
      noise_w.T, noise_b.reshape(_E, 1),
      noise_sample.T)

    expert_idx = idx_t.T.reshape(_B * _K)
    pair_gate = gate_t.T.reshape(_B * _K)

    w1r = conv1_w.transpose(0, 1, 3, 4, 2).reshape(
        _E, _DFF, 9 * _DIM).astype(jnp.bfloat16)
    w2r = conv2_w.transpose(0, 3, 4, 1, 2).reshape(
        _E, 9 * _DIM, _DFF).astype(jnp.bfloat16)

    grid_spec = pltpu.PrefetchScalarGridSpec(
        num_scalar_prefetch=2,
        grid=(_B,),
        in_specs=[
            pl.BlockSpec((1, _DIM, _HW), lambda s, ei, g: (s, 0, 0)),
            pl.BlockSpec((1, _DFF, 9 * _DIM), lambda s, ei, g: (ei[2 * s], 0, 0)),
            pl.BlockSpec((1, _DFF, 9 * _DIM),
                         lambda s, ei, g: (ei[2 * s + 1], 0, 0)),
            pl.BlockSpec((1, 9 * _DIM, _DFF), lambda s, ei, g: (ei[2 * s], 0, 0)),
            pl.BlockSpec((1, 9 * _DIM, _DFF),
                         lambda s, ei, g: (ei[2 * s + 1], 0, 0)),
        ],
        out_specs=pl.BlockSpec((1, _DIM, _HW), lambda s, ei, g: (s, 0, 0)),
    )

    y = pl.pallas_call(
        _dispatch_kernel,
        grid_spec=grid_spec,
        out_shape=jax.ShapeDtypeStruct((_B, _DIM, _HW), jnp.float32),
        compiler_params=pltpu.CompilerParams(
            dimension_semantics=("arbitrary",),
        ),
    )(expert_idx, pair_gate, x_flat, w1r, w1r, w2r, w2r)

    return (y.reshape(_B, _DIM, _H, _W), loss[0, 0])
